# baseline (speedup 1.0000x reference)
import functools

import jax
import jax.numpy as jnp
from jax import lax
from jax.experimental import pallas as pl
from jax.experimental.pallas import tpu as pltpu
from jax.experimental.pallas import tpu_sc as plsc

E = 64
N = 1024
B_TOTAL = 16 * 32 * 32
NHALF = 2
B_HALF = B_TOTAL // NHALF
BLOCK = 2048
NB = B_HALF // BLOCK

_NC, _NS = 2, 16
NW = _NC * _NS
BPW = B_HALF // NW
CHUNK = 128
NCHUNK = BPW // CHUNK


def _tc_body_wt(x_ref, w_ref, idx_ref, wt_ref):
    i = pl.program_id(0)
    xb = x_ref[...]
    w = w_ref[...]
    d = (jnp.sum(xb ** 2, axis=1, keepdims=True)
         - 2.0 * jnp.dot(xb, w)
         + jnp.sum(w ** 2, axis=0, keepdims=True))
    idx = jnp.argmin(d, axis=1).astype(jnp.int32)
    idx_ref[...] = idx.reshape(idx_ref.shape)

    @pl.when(i == 0)
    def _():
        wt_ref[...] = w.T


def _tc_body(x_ref, w_ref, idx_ref):
    xb = x_ref[...]
    w = w_ref[...]
    d = (jnp.sum(xb ** 2, axis=1, keepdims=True)
         - 2.0 * jnp.dot(xb, w)
         + jnp.sum(w ** 2, axis=0, keepdims=True))
    idx = jnp.argmin(d, axis=1).astype(jnp.int32)
    idx_ref[...] = idx.reshape(idx_ref.shape)


def _tc_indices_wt(x2d, w):
    return pl.pallas_call(
        _tc_body_wt,
        grid=(NB,),
        in_specs=[
            pl.BlockSpec((BLOCK, E), lambda i: (i, 0)),
            pl.BlockSpec((E, N), lambda i: (0, 0)),
        ],
        out_specs=[
            pl.BlockSpec((BLOCK // 128, 128), lambda i: (i, 0)),
            pl.BlockSpec((N, E), lambda i: (0, 0)),
        ],
        out_shape=[
            jax.ShapeDtypeStruct((B_HALF // 128, 128), jnp.int32),
            jax.ShapeDtypeStruct((N, E), jnp.float32),
        ],
    )(x2d, w)


def _tc_indices(x2d, w):
    return pl.pallas_call(
        _tc_body,
        grid=(NB,),
        in_specs=[
            pl.BlockSpec((BLOCK, E), lambda i: (i, 0)),
            pl.BlockSpec((E, N), lambda i: (0, 0)),
        ],
        out_specs=pl.BlockSpec((BLOCK // 128, 128), lambda i: (i, 0)),
        out_shape=jax.ShapeDtypeStruct((B_HALF // 128, 128), jnp.int32),
    )(x2d, w)


@functools.cache
def _make_sc_gather():
    mesh = plsc.VectorSubcoreMesh(core_axis_name="c", subcore_axis_name="s")

    @functools.partial(
        pl.kernel,
        mesh=mesh,
        out_type=jax.ShapeDtypeStruct((B_HALF, E), jnp.float32),
        scratch_types=[
            pltpu.VMEM((NCHUNK, CHUNK), jnp.int32),
            pltpu.VMEM((BPW, E), jnp.float32),
            pltpu.SemaphoreType.DMA,
            pltpu.SemaphoreType.DMA,
        ],
        compiler_params=pltpu.CompilerParams(use_tc_tiling_on_sc=False),
    )
    def _sc_gather(table_hbm, idx_hbm, out_hbm, idx_v, rows_v, gsem, wsem):
        wid = lax.axis_index("s") * _NC + lax.axis_index("c")
        pltpu.sync_copy(idx_hbm.at[pl.ds(wid * NCHUNK, NCHUNK)], idx_v)
        gathers = [
            pltpu.async_copy(
                table_hbm.at[idx_v.at[c]],
                rows_v.at[pl.ds(c * CHUNK, CHUNK)],
                gsem)
            for c in range(NCHUNK)
        ]
        writes = []
        for c in range(NCHUNK):
            gathers[c].wait()
            writes.append(pltpu.async_copy(
                rows_v.at[pl.ds(c * CHUNK, CHUNK)],
                out_hbm.at[pl.ds(wid * BPW + c * CHUNK, CHUNK)],
                wsem))
        for cp in writes:
            cp.wait()

    return _sc_gather


def kernel(x, w):
    x2d = x.reshape(B_TOTAL, E)
    sc = _make_sc_gather()
    idx0, wt = _tc_indices_wt(x2d[:B_HALF], w)
    out0 = sc(wt, idx0)
    idx1 = _tc_indices(x2d[B_HALF:], w)
    out1 = sc(wt, idx1)
    out = jnp.concatenate([out0, out1], axis=0)
    return out.reshape(x.shape)


# speedup vs baseline: 1.1030x; 1.1030x over previous
import functools

import jax
import jax.numpy as jnp
from jax import lax
from jax.experimental import pallas as pl
from jax.experimental.pallas import tpu as pltpu
from jax.experimental.pallas import tpu_sc as plsc

E = 64
N = 1024
B_TOTAL = 16 * 32 * 32
BLOCK = 2048
NB = B_TOTAL // BLOCK


def _tc_body(x_ref, w_ref, idx_ref, wt_ref):
    i = pl.program_id(0)
    xb = x_ref[...]
    w = w_ref[...]
    d = (jnp.sum(xb ** 2, axis=1, keepdims=True)
         - 2.0 * jnp.dot(xb, w)
         + jnp.sum(w ** 2, axis=0, keepdims=True))
    idx = jnp.argmin(d, axis=1).astype(jnp.int32)
    idx_ref[...] = idx.reshape(idx_ref.shape)

    @pl.when(i == 0)
    def _():
        wt_ref[...] = w.T


def _tc_indices(x2d, w):
    return pl.pallas_call(
        _tc_body,
        grid=(NB,),
        in_specs=[
            pl.BlockSpec((BLOCK, E), lambda i: (i, 0)),
            pl.BlockSpec((E, N), lambda i: (0, 0)),
        ],
        out_specs=[
            pl.BlockSpec((BLOCK // 128, 128), lambda i: (i, 0)),
            pl.BlockSpec((N, E), lambda i: (0, 0)),
        ],
        out_shape=[
            jax.ShapeDtypeStruct((B_TOTAL // 128, 128), jnp.int32),
            jax.ShapeDtypeStruct((N, E), jnp.float32),
        ],
    )(x2d, w)


_NC, _NS = 2, 16
NW = _NC * _NS
BPW = B_TOTAL // NW
CHUNK = 128
NCHUNK = BPW // CHUNK

@functools.cache
def _make_sc_gather():
    mesh = plsc.VectorSubcoreMesh(core_axis_name="c", subcore_axis_name="s")

    @functools.partial(
        pl.kernel,
        mesh=mesh,
        out_type=jax.ShapeDtypeStruct((B_TOTAL, E), jnp.float32),
        scratch_types=[
            pltpu.VMEM((NCHUNK, CHUNK), jnp.int32),
            pltpu.VMEM((BPW, E), jnp.float32),
            pltpu.SemaphoreType.DMA,
            pltpu.SemaphoreType.DMA,
        ],
        compiler_params=pltpu.CompilerParams(use_tc_tiling_on_sc=False),
    )
    def _sc_gather(table_hbm, idx_hbm, out_hbm, idx_v, rows_v, gsem, wsem):
        wid = lax.axis_index("s") * _NC + lax.axis_index("c")
        pltpu.sync_copy(idx_hbm.at[pl.ds(wid * NCHUNK, NCHUNK)], idx_v)
        gathers = [
            pltpu.async_copy(
                table_hbm.at[idx_v.at[c]],
                rows_v.at[pl.ds(c * CHUNK, CHUNK)],
                gsem)
            for c in range(NCHUNK)
        ]
        writes = []
        for c in range(NCHUNK):
            gathers[c].wait()
            writes.append(pltpu.async_copy(
                rows_v.at[pl.ds(c * CHUNK, CHUNK)],
                out_hbm.at[pl.ds(wid * BPW + c * CHUNK, CHUNK)],
                wsem))
        for cp in writes:
            cp.wait()

    return _sc_gather


def kernel(x, w):
    x2d = x.reshape(B_TOTAL, E)
    idx2d, wt = _tc_indices(x2d, w)
    out = _make_sc_gather()(wt, idx2d)
    return out.reshape(x.shape)


# speedup vs baseline: 1.1546x; 1.0468x over previous
import functools

import jax
import jax.numpy as jnp
from jax import lax
from jax.experimental import pallas as pl
from jax.experimental.pallas import tpu as pltpu
from jax.experimental.pallas import tpu_sc as plsc

E = 64
N = 1024
B_TOTAL = 16 * 32 * 32
BLOCK = 4096
NB = B_TOTAL // BLOCK


def _tc_body(x_ref, w_ref, idx_ref, wt_ref):
    i = pl.program_id(0)
    xb = x_ref[...]
    w = w_ref[...]
    d = (jnp.sum(xb ** 2, axis=1, keepdims=True)
         - 2.0 * jnp.dot(xb, w)
         + jnp.sum(w ** 2, axis=0, keepdims=True))
    idx = jnp.argmin(d, axis=1).astype(jnp.int32)
    idx_ref[...] = idx.reshape(idx_ref.shape)

    @pl.when(i == 0)
    def _():
        wt_ref[...] = w.T


def _tc_indices(x2d, w):
    return pl.pallas_call(
        _tc_body,
        grid=(NB,),
        in_specs=[
            pl.BlockSpec((BLOCK, E), lambda i: (i, 0)),
            pl.BlockSpec((E, N), lambda i: (0, 0)),
        ],
        out_specs=[
            pl.BlockSpec((BLOCK // 128, 128), lambda i: (i, 0)),
            pl.BlockSpec((N, E), lambda i: (0, 0)),
        ],
        out_shape=[
            jax.ShapeDtypeStruct((B_TOTAL // 128, 128), jnp.int32),
            jax.ShapeDtypeStruct((N, E), jnp.float32),
        ],
    )(x2d, w)


_NC, _NS = 2, 16
NW = _NC * _NS
BPW = B_TOTAL // NW
CHUNK = 128
NCHUNK = BPW // CHUNK

@functools.cache
def _make_sc_gather():
    mesh = plsc.VectorSubcoreMesh(core_axis_name="c", subcore_axis_name="s")

    @functools.partial(
        pl.kernel,
        mesh=mesh,
        out_type=jax.ShapeDtypeStruct((B_TOTAL, E), jnp.float32),
        scratch_types=[
            pltpu.VMEM((NCHUNK, CHUNK), jnp.int32),
            pltpu.VMEM((BPW, E), jnp.float32),
            pltpu.SemaphoreType.DMA,
        ],
        compiler_params=pltpu.CompilerParams(use_tc_tiling_on_sc=False),
    )
    def _sc_gather(table_hbm, idx_hbm, out_hbm, idx_v, rows_v, sem):
        wid = lax.axis_index("s") * _NC + lax.axis_index("c")
        pltpu.sync_copy(idx_hbm.at[pl.ds(wid * NCHUNK, NCHUNK)], idx_v)
        copies = []
        for c in range(NCHUNK):
            copies.append(pltpu.async_copy(
                table_hbm.at[idx_v.at[c]],
                rows_v.at[pl.ds(c * CHUNK, CHUNK)],
                sem))
        for cp in copies:
            cp.wait()
        pltpu.sync_copy(rows_v, out_hbm.at[pl.ds(wid * BPW, BPW)])

    return _sc_gather


def kernel(x, w):
    x2d = x.reshape(B_TOTAL, E)
    idx2d, wt = _tc_indices(x2d, w)
    out = _make_sc_gather()(wt, idx2d)
    return out.reshape(x.shape)
